# baseline (speedup 1.0000x reference)
import jax
import jax.numpy as jnp
from jax import lax
from jax.experimental import pallas as pl
from jax.experimental.pallas import tpu as pltpu
from jax.experimental.pallas import tpu_sc as plsc

NC = 2
NS = 16
NW = NC * NS
L = 16

B = 16384
D = 128
BPW = B // NW
CH = 128
NCHUNK = BPW // CH
DEPTH = 2


def _make_sc_kernel():
    mesh = plsc.VectorSubcoreMesh(core_axis_name="c", subcore_axis_name="s")

    @pl.kernel(
        out_type=jax.ShapeDtypeStruct((B,), jnp.float32),
        mesh=mesh,
        compiler_params=pltpu.CompilerParams(needs_layout_passes=False),
        scratch_types=(
            [pltpu.VMEM((BPW,), jnp.int32)] * 2
            + [pltpu.VMEM((DEPTH, CH, D), jnp.float32)] * 2
            + [pltpu.VMEM((BPW + L,), jnp.float32)]
            + [pltpu.SemaphoreType.DMA((DEPTH,))] * 2
            + [pltpu.SemaphoreType.DMA] * 2
        ),
    )
    def sc_dot(uidx_hbm, iidx_hbm, w_hbm, h_hbm, out_hbm,
               uidx_v, iidx_v, ubuf_r, hbuf_r, outbuf,
               sems_u, sems_h, sem_iu, sem_ii):
        wid = lax.axis_index("s") * NC + lax.axis_index("c")
        base = wid * BPW
        cp_iu = pltpu.async_copy(uidx_hbm.at[pl.ds(base, BPW)], uidx_v, sem_iu)
        cp_ii = pltpu.async_copy(iidx_hbm.at[pl.ds(base, BPW)], iidx_v, sem_ii)

        lanes = lax.iota(jnp.int32, L)
        perms = {s: lanes ^ s for s in (1, 2, 4, 8)}
        lane0 = lanes == 0
        dnums = lax.GatherDimensionNumbers(
            offset_dims=(), collapsed_slice_dims=(0,), start_index_map=(0,))

        def _lane_shuffle(v, perm):
            return lax.gather(v, perm.reshape(L, 1), dimension_numbers=dnums,
                              slice_sizes=(1,),
                              mode=lax.GatherScatterMode.PROMISE_IN_BOUNDS)

        def _start_u(c, p):
            return pltpu.async_copy(w_hbm.at[uidx_v.at[pl.ds(c * CH, CH)]],
                                    ubuf_r.at[p], sems_u.at[p])

        def _start_h(c, p):
            return pltpu.async_copy(h_hbm.at[iidx_v.at[pl.ds(c * CH, CH)]],
                                    hbuf_r.at[p], sems_h.at[p])

        def _compute(c, p):
            ubuf = ubuf_r.at[p]
            hbuf = hbuf_r.at[p]
            @plsc.parallel_loop(0, CH, step=1, unroll=1)
            def _row(i):
                urow = ubuf.at[i]
                hrow = hbuf.at[i]
                ps = [urow[pl.ds(k * L, L)] * hrow[pl.ds(k * L, L)]
                      for k in range(D // L)]
                while len(ps) > 1:
                    ps = [ps[i2] + ps[i2 + 1] for i2 in range(0, len(ps), 2)]
                acc = ps[0]
                for s in (1, 2, 4, 8):
                    acc = acc + _lane_shuffle(acc, perms[s])
                plsc.store_compressed(outbuf.at[pl.ds(c * CH + i, L)],
                                      acc, mask=lane0)

        cp_iu.wait()
        for c in range(min(DEPTH, NCHUNK)):
            _start_u(c, c)
        cp_ii.wait()
        for c in range(min(DEPTH, NCHUNK)):
            _start_h(c, c)

        @pl.loop(0, NCHUNK)
        def _chunk(c):
            p = lax.rem(c, DEPTH)
            pltpu.make_async_copy(w_hbm.at[uidx_v.at[pl.ds(0, CH)]],
                                  ubuf_r.at[p], sems_u.at[p]).wait()
            pltpu.make_async_copy(h_hbm.at[iidx_v.at[pl.ds(0, CH)]],
                                  hbuf_r.at[p], sems_h.at[p]).wait()
            _compute(c, p)
            @pl.when(c + DEPTH < NCHUNK)
            def _refill():
                _start_u(c + DEPTH, p)
                _start_h(c + DEPTH, p)

        pltpu.sync_copy(outbuf.at[pl.ds(0, BPW)], out_hbm.at[pl.ds(base, BPW)])

    return sc_dot


_sc_dot = _make_sc_kernel()


def kernel(user_idx, item_idx, W, H):
    y = _sc_dot(user_idx.astype(jnp.int32), item_idx.astype(jnp.int32), W, H)
    return y.reshape(-1, 1)


# speedup vs baseline: 1.0146x; 1.0146x over previous
import jax
import jax.numpy as jnp
from jax import lax
from jax.experimental import pallas as pl
from jax.experimental.pallas import tpu as pltpu
from jax.experimental.pallas import tpu_sc as plsc

NC = 2
NS = 16
NW = NC * NS
L = 16

B = 16384
D = 128
BPW = B // NW
CH = 128
NCHUNK = BPW // CH
DEPTH = 3


def _make_sc_kernel():
    mesh = plsc.VectorSubcoreMesh(core_axis_name="c", subcore_axis_name="s")

    @pl.kernel(
        out_type=jax.ShapeDtypeStruct((B,), jnp.float32),
        mesh=mesh,
        compiler_params=pltpu.CompilerParams(needs_layout_passes=False),
        scratch_types=(
            [pltpu.VMEM((BPW,), jnp.int32)] * 2
            + [pltpu.VMEM((DEPTH, CH, D), jnp.float32)] * 2
            + [pltpu.VMEM((BPW + L,), jnp.float32)]
            + [pltpu.SemaphoreType.DMA((DEPTH,))] * 2
            + [pltpu.SemaphoreType.DMA] * 2
        ),
    )
    def sc_dot(uidx_hbm, iidx_hbm, w_hbm, h_hbm, out_hbm,
               uidx_v, iidx_v, ubuf_r, hbuf_r, outbuf,
               sems_u, sems_h, sem_iu, sem_ii):
        wid = lax.axis_index("s") * NC + lax.axis_index("c")
        base = wid * BPW
        cp_iu = pltpu.async_copy(uidx_hbm.at[pl.ds(base, BPW)], uidx_v, sem_iu)
        cp_ii = pltpu.async_copy(iidx_hbm.at[pl.ds(base, BPW)], iidx_v, sem_ii)

        lanes = lax.iota(jnp.int32, L)
        perms = {s: lanes ^ s for s in (1, 2, 4, 8)}
        lane0 = lanes == 0
        dnums = lax.GatherDimensionNumbers(
            offset_dims=(), collapsed_slice_dims=(0,), start_index_map=(0,))

        def _lane_shuffle(v, perm):
            return lax.gather(v, perm.reshape(L, 1), dimension_numbers=dnums,
                              slice_sizes=(1,),
                              mode=lax.GatherScatterMode.PROMISE_IN_BOUNDS)

        def _start_u(c, p):
            return pltpu.async_copy(w_hbm.at[uidx_v.at[pl.ds(c * CH, CH)]],
                                    ubuf_r.at[p], sems_u.at[p])

        def _start_h(c, p):
            return pltpu.async_copy(h_hbm.at[iidx_v.at[pl.ds(c * CH, CH)]],
                                    hbuf_r.at[p], sems_h.at[p])

        def _compute(c, p):
            ubuf = ubuf_r.at[p]
            hbuf = hbuf_r.at[p]
            @plsc.parallel_loop(0, CH, step=1, unroll=1)
            def _row(i):
                urow = ubuf.at[i]
                hrow = hbuf.at[i]
                ps = [urow[pl.ds(k * L, L)] * hrow[pl.ds(k * L, L)]
                      for k in range(D // L)]
                while len(ps) > 1:
                    ps = [ps[i2] + ps[i2 + 1] for i2 in range(0, len(ps), 2)]
                acc = ps[0]
                for s in (1, 2, 4, 8):
                    acc = acc + _lane_shuffle(acc, perms[s])
                plsc.store_compressed(outbuf.at[pl.ds(c * CH + i, L)],
                                      acc, mask=lane0)

        cp_iu.wait()
        for c in range(min(DEPTH, NCHUNK)):
            _start_u(c, c)
        cp_ii.wait()
        for c in range(min(DEPTH, NCHUNK)):
            _start_h(c, c)

        @pl.loop(0, NCHUNK)
        def _chunk(c):
            p = lax.rem(c, DEPTH)
            pltpu.make_async_copy(w_hbm.at[uidx_v.at[pl.ds(0, CH)]],
                                  ubuf_r.at[p], sems_u.at[p]).wait()
            pltpu.make_async_copy(h_hbm.at[iidx_v.at[pl.ds(0, CH)]],
                                  hbuf_r.at[p], sems_h.at[p]).wait()
            _compute(c, p)
            @pl.when(c + DEPTH < NCHUNK)
            def _refill():
                _start_u(c + DEPTH, p)
                _start_h(c + DEPTH, p)

        pltpu.sync_copy(outbuf.at[pl.ds(0, BPW)], out_hbm.at[pl.ds(base, BPW)])

    return sc_dot


_sc_dot = _make_sc_kernel()


def kernel(user_idx, item_idx, W, H):
    y = _sc_dot(user_idx.astype(jnp.int32), item_idx.astype(jnp.int32), W, H)
    return y.reshape(-1, 1)
